# baseline (speedup 1.0000x reference)
import functools

import jax
import jax.numpy as jnp
from jax import lax
from jax.experimental import pallas as pl
from jax.experimental.pallas import tpu as pltpu
from jax.experimental.pallas import tpu_sc as plsc

_LR = 0.01
_L = 16


def _tc_copy(x):
    V, D = x.shape
    BR = next(b for b in range(8000, 0, -8) if V % b == 0 and b % 8 == 0)

    def cpy(x_ref, o_ref):
        o_ref[...] = x_ref[...]

    return pl.pallas_call(
        cpy,
        grid=(V // BR,),
        in_specs=[pl.BlockSpec((BR, D), lambda i: (i, 0))],
        out_specs=pl.BlockSpec((BR, D), lambda i: (i, 0)),
        out_shape=jax.ShapeDtypeStruct((V, D), jnp.float32),
    )(x)


def _make_update_kernel(V, D, B):
    assert D == 32, "kernel specialized for 32-wide rows"
    NC, NS = 2, 16
    NW = NC * NS
    assert V % NW == 0
    RPW = V // NW
    TBL = ((RPW + _L - 1) // _L) * _L
    ROUND = 4096 if B % 4096 == 0 else B
    NR = B // ROUND
    CAP = ROUND + 128
    CHUNK = 128

    mesh = plsc.VectorSubcoreMesh(
        core_axis_name="c", subcore_axis_name="s", num_cores=NC, num_subcores=NS
    )

    @functools.partial(
        pl.kernel,
        mesh=mesh,
        out_type=(),
        compiler_params=pltpu.CompilerParams(
            needs_layout_passes=False, use_tc_tiling_on_sc=False
        ),
        scratch_types=[
            pltpu.VMEM((ROUND,), jnp.int32),
            pltpu.VMEM((TBL,), jnp.int32),
            pltpu.VMEM((CAP,), jnp.int32),
            pltpu.VMEM((CAP,), jnp.int32),
            pltpu.VMEM((CHUNK,), jnp.int32),
            pltpu.VMEM((CHUNK,), jnp.int32),
            pltpu.VMEM((CHUNK,), jnp.float32),
            pltpu.VMEM((CHUNK, 32), jnp.float32),
            pltpu.VMEM((CHUNK, 32), jnp.float32),
            pltpu.VMEM((CHUNK, 32), jnp.float32),
            pltpu.SemaphoreType.DMA,
            pltpu.SemaphoreType.DMA,
        ],
    )
    def body(param_hbm, gv_hbm, gi_hbm, out_hbm,
             idxbuf, table, oidx, ob, sidx, fbuf, lrbuf,
             prows, grows, orows, sem1, sem2):
        wid = lax.axis_index("s") * NC + lax.axis_index("c")
        lo = wid * RPW
        iota = lax.iota(jnp.int32, _L)

        neg1 = jnp.full((_L,), -1, jnp.int32)
        allt = jnp.full((_L,), True, jnp.bool_)

        def init_body(j, carry):
            plsc.store_scatter(table, [iota + j * _L], neg1, mask=allt)
            return carry

        lax.fori_loop(0, TBL // _L, init_body, 0)

        lov = jnp.full((_L,), 0, jnp.int32) + lo

        def scan_phase(r):
            rbase = r * ROUND
            pltpu.sync_copy(gi_hbm.at[pl.ds(rbase, ROUND)], idxbuf)

            def scan_body(i, off):
                v = idxbuf[pl.ds(i * _L, _L)]
                m = (v >= lo) & (v < lo + RPW)
                mi = jnp.where(m, 1, 0).astype(jnp.int32)
                s = plsc.cumsum(mi)
                pos = s + (off - 1)
                plsc.store_scatter(oidx, [pos], v, mask=m)
                plsc.store_scatter(ob, [pos], iota + (rbase + i * _L), mask=m)
                return off + jnp.sum(mi)

            off = lax.fori_loop(0, ROUND // _L, scan_body, jnp.int32(0))

            for k in range(CHUNK // _L):
                plsc.store_scatter(oidx, [iota + (off + k * _L)], lov, mask=allt)

            def p1_body(j, carry2):
                base = j * _L
                v = plsc.load_gather(oidx, [iota + base])
                b = plsc.load_gather(ob, [iota + base])
                lv = v - lo
                valid = (iota + base) < off
                for l in range(_L):
                    plsc.store_scatter(table, [lv], b, mask=valid & (iota == l))
                return carry2

            nch1 = (off + (_L - 1)) // _L
            lax.fori_loop(0, nch1, p1_body, 0)
            return off

        def update_phase(off):
            def p3_body(c, carry2):
                base = c * CHUNK
                for k in range(CHUNK // _L):
                    idxs = plsc.load_gather(oidx, [iota + (base + k * _L)])
                    sidx[pl.ds(k * _L, _L)] = idxs
                    tb = plsc.load_gather(table, [idxs - lo])
                    fbuf[pl.ds(k * _L, _L)] = jnp.maximum(tb, 0)
                    lrbuf[pl.ds(k * _L, _L)] = jnp.where(
                        tb >= 0, _LR, 0.0
                    ).astype(jnp.float32)
                cp1 = pltpu.async_copy(param_hbm.at[sidx], prows, sem1)
                cp2 = pltpu.async_copy(gv_hbm.at[fbuf], grows, sem2)
                cp1.wait()
                cp2.wait()
                for g in range(CHUNK // _L):
                    rows = iota + g * _L
                    lr16 = lrbuf[pl.ds(g * _L, _L)]
                    for col in range(32):
                        cols = jnp.full((_L,), col, jnp.int32)
                        p = plsc.load_gather(prows, [rows, cols])
                        gv = plsc.load_gather(grows, [rows, cols])
                        plsc.store_scatter(
                            orows, [rows, cols], p - lr16 * gv, mask=allt
                        )
                cp3 = pltpu.async_copy(orows, out_hbm.at[sidx], sem1)
                cp3.wait()
                return carry2

            nch3 = (off + (CHUNK - 1)) // CHUNK
            lax.fori_loop(0, nch3, p3_body, 0)

        def round_body(r, carry):
            update_phase(scan_phase(r))
            return carry

        lax.fori_loop(0, NR, round_body, 0)

    return body


def kernel(param, grad_values, grad_indices):
    V, D = param.shape
    B = grad_values.shape[0]
    out0 = _tc_copy(param)
    out_ref = jax.new_ref(out0)
    upd = _make_update_kernel(V, D, B)
    upd(param, grad_values, grad_indices, out_ref)
    return out_ref[...]


# speedup vs baseline: 1.5840x; 1.5840x over previous
import functools

import jax
import jax.numpy as jnp
from jax import lax
from jax.experimental import pallas as pl
from jax.experimental.pallas import tpu as pltpu
from jax.experimental.pallas import tpu_sc as plsc

_LR = 0.01
_L = 16


def _make_update_kernel(V, D, B):
    assert D == 32, "kernel specialized for 32-wide rows"
    NC, NS = 2, 16
    NW = NC * NS
    assert V % NW == 0
    RPW = V // NW
    TBL = ((RPW + _L - 1) // _L) * _L
    ROUND = 4096 if B % 4096 == 0 else B
    NR = B // ROUND
    CAP = ROUND + 128
    CHUNK = 128
    CR = next(c for c in range(min(625, RPW), 0, -1) if RPW % c == 0)
    NCH = RPW // CR
    NB = 3
    LOOKAHEAD = 2

    mesh = plsc.VectorSubcoreMesh(
        core_axis_name="c", subcore_axis_name="s", num_cores=NC, num_subcores=NS
    )

    @functools.partial(
        pl.kernel,
        mesh=mesh,
        out_type=jax.ShapeDtypeStruct((V, D), jnp.float32),
        compiler_params=pltpu.CompilerParams(
            needs_layout_passes=False, use_tc_tiling_on_sc=False
        ),
        scratch_types=[
            pltpu.VMEM((ROUND,), jnp.int32),
            pltpu.VMEM((TBL,), jnp.int32),
            pltpu.VMEM((CAP,), jnp.int32),
            pltpu.VMEM((CAP,), jnp.int32),
            pltpu.VMEM((CHUNK,), jnp.int32),
            pltpu.VMEM((CHUNK,), jnp.int32),
            pltpu.VMEM((CHUNK,), jnp.float32),
            pltpu.VMEM((CHUNK, 32), jnp.float32),
            pltpu.VMEM((CHUNK, 32), jnp.float32),
            pltpu.VMEM((CHUNK, 32), jnp.float32),
            pltpu.VMEM((NB, 625, 32), jnp.float32),
            pltpu.SemaphoreType.DMA,
            pltpu.SemaphoreType.DMA,
            [pltpu.SemaphoreType.DMA] * NB,
            [pltpu.SemaphoreType.DMA] * NB,
        ],
    )
    def body(param_hbm, gv_hbm, gi_hbm, out_hbm,
             idxbuf, table, oidx, ob, sidx, fbuf, lrbuf,
             prows, grows, orows, ring, sem1, sem2, lsems, ssems):
        wid = lax.axis_index("s") * NC + lax.axis_index("c")
        lo = wid * RPW
        iota = lax.iota(jnp.int32, _L)

        def start_load(b, c):
            return pltpu.async_copy(
                param_hbm.at[pl.ds(lo + c * CR, CR)], ring.at[b, pl.ds(0, CR)],
                lsems[b]
            )

        def start_store(b, c):
            return pltpu.async_copy(
                ring.at[b, pl.ds(0, CR)], out_hbm.at[pl.ds(lo + c * CR, CR)],
                ssems[b]
            )

        loads = [None] * NB
        stores = [None] * NB
        for b in range(min(LOOKAHEAD, NCH)):
            loads[b] = start_load(b, b)
        for c in range(NCH):
            b = c % NB
            loads[b].wait()
            loads[b] = None
            f = c + LOOKAHEAD
            if f < NCH:
                fb = f % NB
                if stores[fb] is not None:
                    stores[fb].wait()
                    stores[fb] = None
                loads[fb] = start_load(fb, f)
            stores[b] = start_store(b, c)

        neg1 = jnp.full((_L,), -1, jnp.int32)
        allt = jnp.full((_L,), True, jnp.bool_)

        def init_body(j, carry):
            plsc.store_scatter(table, [iota + j * _L], neg1, mask=allt)
            return carry

        lax.fori_loop(0, TBL // _L, init_body, 0)

        lov = jnp.full((_L,), 0, jnp.int32) + lo

        def scan_phase(r):
            rbase = r * ROUND
            pltpu.sync_copy(gi_hbm.at[pl.ds(rbase, ROUND)], idxbuf)

            def scan_body(i, off):
                v = idxbuf[pl.ds(i * _L, _L)]
                m = (v >= lo) & (v < lo + RPW)
                mi = jnp.where(m, 1, 0).astype(jnp.int32)
                s = plsc.cumsum(mi)
                pos = s + (off - 1)
                plsc.store_scatter(oidx, [pos], v, mask=m)
                plsc.store_scatter(ob, [pos], iota + (rbase + i * _L), mask=m)
                return off + jnp.sum(mi)

            off = lax.fori_loop(0, ROUND // _L, scan_body, jnp.int32(0))

            for k in range(CHUNK // _L):
                plsc.store_scatter(oidx, [iota + (off + k * _L)], lov, mask=allt)

            def p1_body(j, carry2):
                base = j * _L
                v = plsc.load_gather(oidx, [iota + base])
                b = plsc.load_gather(ob, [iota + base])
                lv = v - lo
                valid = (iota + base) < off
                for l in range(_L):
                    plsc.store_scatter(table, [lv], b, mask=valid & (iota == l))
                return carry2

            nch1 = (off + (_L - 1)) // _L
            lax.fori_loop(0, nch1, p1_body, 0)
            return off

        def update_phase(off):
            def p3_body(c, carry2):
                base = c * CHUNK
                for k in range(CHUNK // _L):
                    idxs = plsc.load_gather(oidx, [iota + (base + k * _L)])
                    sidx[pl.ds(k * _L, _L)] = idxs
                    tb = plsc.load_gather(table, [idxs - lo])
                    fbuf[pl.ds(k * _L, _L)] = jnp.maximum(tb, 0)
                    lrbuf[pl.ds(k * _L, _L)] = jnp.where(
                        tb >= 0, _LR, 0.0
                    ).astype(jnp.float32)
                cp1 = pltpu.async_copy(param_hbm.at[sidx], prows, sem1)
                cp2 = pltpu.async_copy(gv_hbm.at[fbuf], grows, sem2)
                cp1.wait()
                cp2.wait()
                for g in range(CHUNK // _L):
                    rows = iota + g * _L
                    lr16 = lrbuf[pl.ds(g * _L, _L)]
                    for col in range(32):
                        cols = jnp.full((_L,), col, jnp.int32)
                        p = plsc.load_gather(prows, [rows, cols])
                        gv = plsc.load_gather(grows, [rows, cols])
                        plsc.store_scatter(
                            orows, [rows, cols], p - lr16 * gv, mask=allt
                        )
                cp3 = pltpu.async_copy(orows, out_hbm.at[sidx], sem1)
                cp3.wait()
                return carry2

            nch3 = (off + (CHUNK - 1)) // CHUNK
            lax.fori_loop(0, nch3, p3_body, 0)

        off0 = scan_phase(0)
        for b in range(NB):
            if stores[b] is not None:
                stores[b].wait()
                stores[b] = None
        update_phase(off0)

        def round_body(r, carry):
            update_phase(scan_phase(r))
            return carry

        lax.fori_loop(1, NR, round_body, 0)

    return body


def kernel(param, grad_values, grad_indices):
    V, D = param.shape
    B = grad_values.shape[0]
    upd = _make_update_kernel(V, D, B)
    return upd(param, grad_values, grad_indices)
